# baseline (speedup 1.0000x reference)
import functools

import jax
import jax.numpy as jnp
from jax import lax
from jax.experimental import pallas as pl
from jax.experimental.pallas import tpu as pltpu
from jax.experimental.pallas import tpu_sc as plsc

N_NODES = 10000
N_EDGES = 320000
F = 128
N_LAYERS = 6
N_GRAPHS = 16
EPS = 1e-7

NC = 2
NS = 16
CW = 128
NBUF = 2
EDGES_PT = -(-N_EDGES // (NS * CW * NBUF)) * CW * NBUF
NCHUNK = EDGES_PT // CW
E_PAD = EDGES_PT * NS
ROWS_PAD = 10240
SLAB = ROWS_PAD // NS

BR = 2000
NB = N_NODES // BR
HF = F // 2


def _ln(z, g, b, eps=1e-5):
    m = jnp.mean(z, axis=-1, keepdims=True)
    v = jnp.mean((z - m) ** 2, axis=-1, keepdims=True)
    return (z - m) * lax.rsqrt(v + eps) * g + b


def _enc_body(x_ref, w_ref, b_ref, o_ref):
    o_ref[...] = (
        jnp.dot(x_ref[...], w_ref[...], preferred_element_type=jnp.float32)
        + b_ref[...]
    )


def _encode(x, W0, b0):
    return pl.pallas_call(
        _enc_body,
        grid=(NB,),
        in_specs=[
            pl.BlockSpec((BR, F), lambda i: (i, 0)),
            pl.BlockSpec((F, F), lambda i: (0, 0)),
            pl.BlockSpec((1, F), lambda i: (0, 0)),
        ],
        out_specs=pl.BlockSpec((BR, F), lambda i: (i, 0)),
        out_shape=jax.ShapeDtypeStruct((N_NODES, F), jnp.float32),
    )(x, W0, b0.reshape(1, F))


def _pre_body(h_ref, g_ref, b_ref, t_ref, hn_ref, p_ref, e_ref):
    hn = jnp.maximum(_ln(h_ref[...], g_ref[...], b_ref[...]), 0.0)
    hn_ref[...] = hn
    m = hn + EPS
    e = jnp.exp(m * t_ref[0, 0])
    e_ref[...] = e
    p_ref[...] = m * e


def _pre(h, g, b, t):
    return pl.pallas_call(
        _pre_body,
        grid=(NB,),
        in_specs=[
            pl.BlockSpec((BR, F), lambda i: (i, 0)),
            pl.BlockSpec((1, F), lambda i: (0, 0)),
            pl.BlockSpec((1, F), lambda i: (0, 0)),
            pl.BlockSpec((1, 1), lambda i: (0, 0)),
        ],
        out_specs=[pl.BlockSpec((BR, F), lambda i: (i, 0))] * 3,
        out_shape=[jax.ShapeDtypeStruct((N_NODES, F), jnp.float32)] * 3,
    )(h, g.reshape(1, F), b.reshape(1, F), t.reshape(1, 1))


def _post_body(num0_ref, num1_ref, den0_ref, den1_ref, hn_ref, h_ref,
               w1_ref, b1_ref, mg_ref, mb_ref, w2_ref, b2_ref, o_ref):
    num = jnp.concatenate([num0_ref[...], num1_ref[...]], axis=1)
    den = jnp.concatenate([den0_ref[...], den1_ref[...]], axis=1)
    aggr = jnp.where(den > 0.0, num / den, 0.0)
    out = aggr + hn_ref[...]
    z = jnp.dot(out, w1_ref[...], preferred_element_type=jnp.float32) + b1_ref[...]
    z = jnp.maximum(_ln(z, mg_ref[...], mb_ref[...]), 0.0)
    o_ref[...] = (
        jnp.dot(z, w2_ref[...], preferred_element_type=jnp.float32)
        + b2_ref[...]
        + h_ref[...]
    )


def _post(num0, num1, den0, den1, hn, h, W1, b1, mg, mb, W2, b2):
    return pl.pallas_call(
        _post_body,
        grid=(NB,),
        in_specs=[
            pl.BlockSpec((BR, HF), lambda i: (i, 0)),
            pl.BlockSpec((BR, HF), lambda i: (i, 0)),
            pl.BlockSpec((BR, HF), lambda i: (i, 0)),
            pl.BlockSpec((BR, HF), lambda i: (i, 0)),
            pl.BlockSpec((BR, F), lambda i: (i, 0)),
            pl.BlockSpec((BR, F), lambda i: (i, 0)),
            pl.BlockSpec((F, 2 * F), lambda i: (0, 0)),
            pl.BlockSpec((1, 2 * F), lambda i: (0, 0)),
            pl.BlockSpec((1, 2 * F), lambda i: (0, 0)),
            pl.BlockSpec((1, 2 * F), lambda i: (0, 0)),
            pl.BlockSpec((2 * F, F), lambda i: (0, 0)),
            pl.BlockSpec((1, F), lambda i: (0, 0)),
        ],
        out_specs=pl.BlockSpec((BR, F), lambda i: (i, 0)),
        out_shape=jax.ShapeDtypeStruct((N_NODES, F), jnp.float32),
    )(num0, num1, den0, den1, hn, h, W1, b1.reshape(1, 2 * F),
      mg.reshape(1, 2 * F), mb.reshape(1, 2 * F), W2, b2.reshape(1, F))


def _pool_body(h_ref, bat_ref, o_ref):
    @pl.when(pl.program_id(0) == 0)
    def _():
        o_ref[...] = jnp.zeros_like(o_ref)

    onehot = (bat_ref[...] ==
              lax.broadcasted_iota(jnp.int32, (1, N_GRAPHS), 1)).astype(jnp.float32)
    o_ref[...] += lax.dot_general(
        onehot, h_ref[...], (((0,), (0,)), ((), ())),
        preferred_element_type=jnp.float32)


def _pool(h, batch):
    return pl.pallas_call(
        _pool_body,
        grid=(NB,),
        in_specs=[
            pl.BlockSpec((BR, F), lambda i: (i, 0)),
            pl.BlockSpec((BR, 1), lambda i: (i, 0)),
        ],
        out_specs=pl.BlockSpec((N_GRAPHS, F), lambda i: (0, 0)),
        out_shape=jax.ShapeDtypeStruct((N_GRAPHS, F), jnp.float32),
    )(h, batch.reshape(N_NODES, 1))


def _sc_body(p_hbm, e_hbm, srcA_hbm, srcB_hbm, dst_hbm,
             num0_out, num1_out, den0_out, den1_out,
             srcA_v, srcB_v, dst_v, rows_v, zbuf, acc, *sems):
    c = lax.axis_index("c")
    s = lax.axis_index("s")
    gs = sems[:NBUF]
    ss = sems[NBUF:]

    pltpu.sync_copy(srcA_hbm.at[s], srcA_v)
    pltpu.sync_copy(srcB_hbm.at[s], srcB_v)
    pltpu.sync_copy(dst_hbm.at[s], dst_v)

    for r in range(16):
        for q in range(HF // 16):
            zbuf[r, pl.ds(q * 16, 16)] = jnp.zeros((16,), jnp.float32)

    def stage(tab, src_v, out_ref):
        @pl.loop(0, 544 // 16)
        def _(j):
            pltpu.sync_copy(zbuf, acc.at[pl.ds(s * 544 + j * 16, 16)])

        plsc.subcore_barrier()

        for b in range(NBUF):
            pltpu.async_copy(tab.at[src_v.at[b]], rows_v.at[b], gs[b])

        @pl.loop(0, NCHUNK // NBUF)
        def _(q):
            for b in range(NBUF):
                j = q * NBUF + b
                pltpu.make_async_copy(tab.at[src_v.at[j]], rows_v.at[b],
                                      gs[b]).wait()

                @pl.when(q < NCHUNK // NBUF - 1)
                def _():
                    pltpu.async_copy(tab.at[src_v.at[j + NBUF]], rows_v.at[b],
                                     gs[b])

        plsc.subcore_barrier()

        pltpu.sync_copy(acc.at[pl.ds(s * 544, 544)],
                        out_ref.at[pl.ds(s * 544, 544)])

    @pl.when(c == 0)
    def _():
        stage(p_hbm, srcA_v, num0_out)
        stage(p_hbm, srcB_v, num1_out)

    @pl.when(c == 1)
    def _():
        stage(e_hbm, srcA_v, den0_out)
        stage(e_hbm, srcB_v, den1_out)


@functools.cache
def _build_sc_aggregate():
    return pl.kernel(
        _sc_body,
        out_type=[jax.ShapeDtypeStruct((ROWS_PAD, HF), jnp.float32)] * 4,
        mesh=plsc.VectorSubcoreMesh(core_axis_name="c", subcore_axis_name="s",
                                    num_cores=NC, num_subcores=NS),
        scratch_types=[
            pltpu.VMEM((NCHUNK, CW), jnp.int32),
            pltpu.VMEM((NCHUNK, CW), jnp.int32),
            pltpu.VMEM((NCHUNK, CW), jnp.int32),
            pltpu.VMEM((NBUF, CW, F), jnp.float32),
            pltpu.VMEM((16, HF), jnp.float32),
            pltpu.VMEM_SHARED((8704, HF), jnp.float32),
        ] + [pltpu.SemaphoreType.DMA] * (2 * NBUF),
        compiler_params=pltpu.CompilerParams(use_tc_tiling_on_sc=False),
    )


def _sc_aggregate(p, e, srcA, srcB, dstI):
    return _build_sc_aggregate()(p, e, srcA, srcB, dstI)


def kernel(x, edge_index, batch, W0, b0, ln_g, ln_b, t, W1, b1, mg, mb,
           W2, b2):
    src = edge_index[0].astype(jnp.int32)
    dst = edge_index[1].astype(jnp.int32)
    src2 = jnp.pad(src, (0, E_PAD - N_EDGES))
    srcA = src2.reshape(NS, NCHUNK, CW)
    srcB = src2.reshape(NS, NCHUNK, CW)
    dstI = jnp.pad(dst, (0, E_PAD - N_EDGES),
                   constant_values=N_NODES).reshape(NS, NCHUNK, CW)

    h = _encode(x, W0, b0)
    for i in range(N_LAYERS):
        hn, p, e = _pre(h, ln_g[i], ln_b[i], t[i])
        num0, num1, den0, den1 = _sc_aggregate(p, e, srcA, srcB, dstI)
        h = _post(num0, num1, den0, den1, hn, h,
                  W1[i], b1[i], mg[i], mb[i], W2[i], b2[i])
    return _pool(h, batch.astype(jnp.int32))


# speedup vs baseline: 1.2786x; 1.2786x over previous
import functools

import jax
import jax.numpy as jnp
from jax import lax
from jax.experimental import pallas as pl
from jax.experimental.pallas import tpu as pltpu
from jax.experimental.pallas import tpu_sc as plsc

N_NODES = 10000
N_EDGES = 320000
F = 128
N_LAYERS = 6
N_GRAPHS = 16
EPS = 1e-7

NC = 2
NS = 16
CW = 128
EDGES_PT = -(-N_EDGES // (NS * CW)) * CW
NCHUNK = EDGES_PT // CW
E_PAD = EDGES_PT * NS
CAP = NCHUNK * CW

SPLIT = N_NODES // 2
ACC_ROWS = 5120
SLAB = ACC_ROWS // NS
LAST_SLAB = SPLIT - (NS - 1) * SLAB

BR = 2000
NB = N_NODES // BR


def _ln(z, g, b, eps=1e-5):
    m = jnp.mean(z, axis=-1, keepdims=True)
    v = jnp.mean((z - m) ** 2, axis=-1, keepdims=True)
    return (z - m) * lax.rsqrt(v + eps) * g + b


def _enc_body(x_ref, w_ref, b_ref, o_ref):
    o_ref[...] = (
        jnp.dot(x_ref[...], w_ref[...], preferred_element_type=jnp.float32)
        + b_ref[...]
    )


def _encode(x, W0, b0):
    return pl.pallas_call(
        _enc_body,
        grid=(NB,),
        in_specs=[
            pl.BlockSpec((BR, F), lambda i: (i, 0)),
            pl.BlockSpec((F, F), lambda i: (0, 0)),
            pl.BlockSpec((1, F), lambda i: (0, 0)),
        ],
        out_specs=pl.BlockSpec((BR, F), lambda i: (i, 0)),
        out_shape=jax.ShapeDtypeStruct((N_NODES, F), jnp.float32),
    )(x, W0, b0.reshape(1, F))


def _pre_body(h_ref, g_ref, b_ref, t_ref, hn_ref, p_ref, e_ref):
    hn = jnp.maximum(_ln(h_ref[...], g_ref[...], b_ref[...]), 0.0)
    hn_ref[...] = hn
    m = hn + EPS
    e = jnp.exp(m * t_ref[0, 0])
    e_ref[...] = e
    p_ref[...] = m * e


def _pre(h, g, b, t):
    return pl.pallas_call(
        _pre_body,
        grid=(NB,),
        in_specs=[
            pl.BlockSpec((BR, F), lambda i: (i, 0)),
            pl.BlockSpec((1, F), lambda i: (0, 0)),
            pl.BlockSpec((1, F), lambda i: (0, 0)),
            pl.BlockSpec((1, 1), lambda i: (0, 0)),
        ],
        out_specs=[pl.BlockSpec((BR, F), lambda i: (i, 0))] * 3,
        out_shape=[jax.ShapeDtypeStruct((N_NODES, F), jnp.float32)] * 3,
    )(h, g.reshape(1, F), b.reshape(1, F), t.reshape(1, 1))


def _post_body(num_ref, den_ref, hn_ref, h_ref, w1_ref, b1_ref, mg_ref,
               mb_ref, w2_ref, b2_ref, o_ref):
    den = den_ref[...]
    aggr = jnp.where(den > 0.0, num_ref[...] / den, 0.0)
    out = aggr + hn_ref[...]
    z = jnp.dot(out, w1_ref[...], preferred_element_type=jnp.float32) + b1_ref[...]
    z = jnp.maximum(_ln(z, mg_ref[...], mb_ref[...]), 0.0)
    o_ref[...] = (
        jnp.dot(z, w2_ref[...], preferred_element_type=jnp.float32)
        + b2_ref[...]
        + h_ref[...]
    )


def _post(num, den, hn, h, W1, b1, mg, mb, W2, b2):
    return pl.pallas_call(
        _post_body,
        grid=(NB,),
        in_specs=[
            pl.BlockSpec((BR, F), lambda i: (i, 0)),
            pl.BlockSpec((BR, F), lambda i: (i, 0)),
            pl.BlockSpec((BR, F), lambda i: (i, 0)),
            pl.BlockSpec((BR, F), lambda i: (i, 0)),
            pl.BlockSpec((F, 2 * F), lambda i: (0, 0)),
            pl.BlockSpec((1, 2 * F), lambda i: (0, 0)),
            pl.BlockSpec((1, 2 * F), lambda i: (0, 0)),
            pl.BlockSpec((1, 2 * F), lambda i: (0, 0)),
            pl.BlockSpec((2 * F, F), lambda i: (0, 0)),
            pl.BlockSpec((1, F), lambda i: (0, 0)),
        ],
        out_specs=pl.BlockSpec((BR, F), lambda i: (i, 0)),
        out_shape=jax.ShapeDtypeStruct((N_NODES, F), jnp.float32),
    )(num, den, hn, h, W1, b1.reshape(1, 2 * F), mg.reshape(1, 2 * F),
      mb.reshape(1, 2 * F), W2, b2.reshape(1, F))


def _pool_body(h_ref, bat_ref, o_ref):
    @pl.when(pl.program_id(0) == 0)
    def _():
        o_ref[...] = jnp.zeros_like(o_ref)

    onehot = (bat_ref[...] ==
              lax.broadcasted_iota(jnp.int32, (1, N_GRAPHS), 1)).astype(jnp.float32)
    o_ref[...] += lax.dot_general(
        onehot, h_ref[...], (((0,), (0,)), ((), ())),
        preferred_element_type=jnp.float32)


def _pool(h, batch):
    return pl.pallas_call(
        _pool_body,
        grid=(NB,),
        in_specs=[
            pl.BlockSpec((BR, F), lambda i: (i, 0)),
            pl.BlockSpec((BR, 1), lambda i: (i, 0)),
        ],
        out_specs=pl.BlockSpec((N_GRAPHS, F), lambda i: (0, 0)),
        out_shape=jax.ShapeDtypeStruct((N_GRAPHS, F), jnp.float32),
    )(h, batch.reshape(N_NODES, 1))


def _part_body(src_hbm, dst_hbm, srcP_out, dstP_out, cnt_out,
               in_s, in_d, ob_s0, ob_d0, ob_s1, ob_d1, cnt_v, sem):
    c = lax.axis_index("c")
    s = lax.axis_index("s")

    @pl.when(c == 0)
    def _():
        zeros = jnp.zeros((16,), jnp.int32)
        sink = jnp.full((16,), SPLIT, jnp.int32)

        @pl.loop(0, CAP // 16)
        def _(r):
            ob_s0[pl.ds(r * 16, 16)] = zeros
            ob_s1[pl.ds(r * 16, 16)] = zeros
            ob_d0[pl.ds(r * 16, 16)] = sink
            ob_d1[pl.ds(r * 16, 16)] = sink

        def chunk(ch, carry):
            pltpu.sync_copy(src_hbm.at[s, ch], in_s)
            pltpu.sync_copy(dst_hbm.at[s, ch], in_d)
            o0, o1 = carry
            for q in range(CW // 16):
                dvec = in_d[pl.ds(q * 16, 16)]
                svec = in_s[pl.ds(q * 16, 16)]
                m0 = dvec < SPLIT
                n0 = plsc.all_reduce_population_count(m0)[0]
                plsc.store_compressed(ob_d0.at[pl.ds(o0, 16)], dvec, mask=m0)
                plsc.store_compressed(ob_s0.at[pl.ds(o0, 16)], svec, mask=m0)
                m1 = jnp.logical_not(m0)
                plsc.store_compressed(ob_d1.at[pl.ds(o1, 16)],
                                      dvec - SPLIT, mask=m1)
                plsc.store_compressed(ob_s1.at[pl.ds(o1, 16)], svec, mask=m1)
                o0 = o0 + n0
                o1 = o1 + (16 - n0)
            return o0, o1

        o0, o1 = pl.loop(0, NCHUNK,
                         init_carry=(jnp.int32(0), jnp.int32(0)))(chunk)

        lane = lax.iota(jnp.int32, 16)
        nch0 = (o0 + CW - 1) // CW
        nch1 = (o1 + CW - 1) // CW
        cnt_v[...] = jnp.where(lane == 0, nch0,
                               jnp.where(lane == 1, nch1, 0))
        pltpu.sync_copy(cnt_v, cnt_out.at[s])

        pltpu.sync_copy(ob_s0, srcP_out.at[s, 0])
        pltpu.sync_copy(ob_s1, srcP_out.at[s, 1])
        pltpu.sync_copy(ob_d0, dstP_out.at[s, 0])
        pltpu.sync_copy(ob_d1, dstP_out.at[s, 1])


@functools.cache
def _build_partition():
    return pl.kernel(
        _part_body,
        out_type=[
            jax.ShapeDtypeStruct((NS, 2, CAP), jnp.int32),
            jax.ShapeDtypeStruct((NS, 2, CAP), jnp.int32),
            jax.ShapeDtypeStruct((NS, 16), jnp.int32),
        ],
        mesh=plsc.VectorSubcoreMesh(core_axis_name="c", subcore_axis_name="s",
                                    num_cores=NC, num_subcores=NS),
        scratch_types=[
            pltpu.VMEM((CW,), jnp.int32),
            pltpu.VMEM((CW,), jnp.int32),
            pltpu.VMEM((CAP,), jnp.int32),
            pltpu.VMEM((CAP,), jnp.int32),
            pltpu.VMEM((CAP,), jnp.int32),
            pltpu.VMEM((CAP,), jnp.int32),
            pltpu.VMEM((16,), jnp.int32),
            pltpu.SemaphoreType.DMA,
        ],
        compiler_params=pltpu.CompilerParams(use_tc_tiling_on_sc=False,
                                             needs_layout_passes=False),
    )


def _agg_body(p_hbm, e_hbm, srcP_hbm, dstP_hbm, cnt_hbm, num_out, den_out,
              src_v, dst_v, rows_v, zbuf, cnt_v, acc, gsem):
    c = lax.axis_index("c")
    s = lax.axis_index("s")

    pltpu.sync_copy(cnt_hbm.at[s], cnt_v)

    for r in range(16):
        for q in range(F // 16):
            zbuf[r, pl.ds(q * 16, 16)] = jnp.zeros((16,), jnp.float32)

    def half(h, tab, out_ref):
        pltpu.sync_copy(srcP_hbm.at[s, h], src_v)
        pltpu.sync_copy(dstP_hbm.at[s, h], dst_v)

        @pl.loop(0, SLAB // 16)
        def _(j):
            pltpu.sync_copy(zbuf, acc.at[pl.ds(s * SLAB + j * 16, 16)])

        plsc.subcore_barrier()

        nch = cnt_v[...][h]

        @pl.loop(0, nch)
        def _(j):
            pltpu.async_copy(tab.at[src_v.at[j]], rows_v, gsem).wait()
            pltpu.sync_copy(rows_v, acc.at[dst_v.at[j]], add=True)

        plsc.subcore_barrier()

        @pl.when(s < NS - 1)
        def _():
            pltpu.sync_copy(
                acc.at[pl.ds(s * SLAB, SLAB)],
                out_ref.at[pl.ds(h * SPLIT + s * SLAB, SLAB)])

        @pl.when(s == NS - 1)
        def _():
            pltpu.sync_copy(
                acc.at[pl.ds((NS - 1) * SLAB, LAST_SLAB)],
                out_ref.at[pl.ds(h * SPLIT + (NS - 1) * SLAB, LAST_SLAB)])

    def run(tab, out_ref):
        half(0, tab, out_ref)
        half(1, tab, out_ref)

    @pl.when(c == 0)
    def _():
        run(p_hbm, num_out)

    @pl.when(c == 1)
    def _():
        run(e_hbm, den_out)


@functools.cache
def _build_aggregate():
    return pl.kernel(
        _agg_body,
        out_type=[jax.ShapeDtypeStruct((N_NODES, F), jnp.float32)] * 2,
        mesh=plsc.VectorSubcoreMesh(core_axis_name="c", subcore_axis_name="s",
                                    num_cores=NC, num_subcores=NS),
        scratch_types=[
            pltpu.VMEM((NCHUNK, CW), jnp.int32),
            pltpu.VMEM((NCHUNK, CW), jnp.int32),
            pltpu.VMEM((CW, F), jnp.float32),
            pltpu.VMEM((16, F), jnp.float32),
            pltpu.VMEM((16,), jnp.int32),
            pltpu.VMEM_SHARED((ACC_ROWS, F), jnp.float32),
            pltpu.SemaphoreType.DMA,
        ],
        compiler_params=pltpu.CompilerParams(use_tc_tiling_on_sc=False),
    )


def _sc_partition(srcI, dstI):
    return _build_partition()(srcI, dstI)


def _sc_aggregate(p, e, srcP, dstP, cnts):
    return _build_aggregate()(p, e, srcP, dstP, cnts)


def kernel(x, edge_index, batch, W0, b0, ln_g, ln_b, t, W1, b1, mg, mb,
           W2, b2):
    src = edge_index[0].astype(jnp.int32)
    dst = edge_index[1].astype(jnp.int32)
    srcI = jnp.pad(src, (0, E_PAD - N_EDGES)).reshape(NS, NCHUNK, CW)
    dstI = jnp.pad(dst, (0, E_PAD - N_EDGES),
                   constant_values=2 * SPLIT).reshape(NS, NCHUNK, CW)

    srcP, dstP, cnts = _sc_partition(srcI, dstI)
    srcP = srcP.reshape(NS, 2, NCHUNK, CW)
    dstP = dstP.reshape(NS, 2, NCHUNK, CW)

    h = _encode(x, W0, b0)
    for i in range(N_LAYERS):
        hn, p, e = _pre(h, ln_g[i], ln_b[i], t[i])
        num, den = _sc_aggregate(p, e, srcP, dstP, cnts)
        h = _post(num, den, hn, h, W1[i], b1[i], mg[i], mb[i], W2[i], b2[i])
    return _pool(h, batch.astype(jnp.int32))


# speedup vs baseline: 1.8038x; 1.4107x over previous
import functools

import jax
import jax.numpy as jnp
from jax import lax
from jax.experimental import pallas as pl
from jax.experimental.pallas import tpu as pltpu
from jax.experimental.pallas import tpu_sc as plsc

N_NODES = 10000
N_EDGES = 320000
F = 128
N_LAYERS = 6
N_GRAPHS = 16
EPS = 1e-7

NC = 2
NS = 16
CW = 96
EDGES_PT = -(-N_EDGES // (NS * CW)) * CW
NCHUNK = EDGES_PT // CW
E_PAD = EDGES_PT * NS
CAP = NCHUNK * CW

SPLIT = N_NODES // 2
ACC_ROWS = 5120
SLAB = ACC_ROWS // NS
LAST_SLAB = SPLIT - (NS - 1) * SLAB

BR = 2000
NB = N_NODES // BR


def _ln(z, g, b, eps=1e-5):
    m = jnp.mean(z, axis=-1, keepdims=True)
    v = jnp.mean((z - m) ** 2, axis=-1, keepdims=True)
    return (z - m) * lax.rsqrt(v + eps) * g + b


def _enc_body(x_ref, w_ref, b_ref, o_ref):
    o_ref[...] = (
        jnp.dot(x_ref[...], w_ref[...], preferred_element_type=jnp.float32)
        + b_ref[...]
    )


def _encode(x, W0, b0):
    return pl.pallas_call(
        _enc_body,
        grid=(NB,),
        in_specs=[
            pl.BlockSpec((BR, F), lambda i: (i, 0)),
            pl.BlockSpec((F, F), lambda i: (0, 0)),
            pl.BlockSpec((1, F), lambda i: (0, 0)),
        ],
        out_specs=pl.BlockSpec((BR, F), lambda i: (i, 0)),
        out_shape=jax.ShapeDtypeStruct((N_NODES, F), jnp.float32),
    )(x, W0, b0.reshape(1, F))


def _pre_body(h_ref, g_ref, b_ref, t_ref, hn_ref, p_ref, e_ref):
    hn = jnp.maximum(_ln(h_ref[...], g_ref[...], b_ref[...]), 0.0)
    hn_ref[...] = hn
    m = hn + EPS
    e = jnp.exp(m * t_ref[0, 0])
    e_ref[...] = e
    p_ref[...] = m * e


def _pre(h, g, b, t):
    return pl.pallas_call(
        _pre_body,
        grid=(NB,),
        in_specs=[
            pl.BlockSpec((BR, F), lambda i: (i, 0)),
            pl.BlockSpec((1, F), lambda i: (0, 0)),
            pl.BlockSpec((1, F), lambda i: (0, 0)),
            pl.BlockSpec((1, 1), lambda i: (0, 0)),
        ],
        out_specs=[pl.BlockSpec((BR, F), lambda i: (i, 0))] * 3,
        out_shape=[jax.ShapeDtypeStruct((N_NODES, F), jnp.float32)] * 3,
    )(h, g.reshape(1, F), b.reshape(1, F), t.reshape(1, 1))


def _post_body(num_ref, den_ref, hn_ref, h_ref, w1_ref, b1_ref, mg_ref,
               mb_ref, w2_ref, b2_ref, o_ref):
    den = den_ref[...]
    aggr = jnp.where(den > 0.0, num_ref[...] / den, 0.0)
    out = aggr + hn_ref[...]
    z = jnp.dot(out, w1_ref[...], preferred_element_type=jnp.float32) + b1_ref[...]
    z = jnp.maximum(_ln(z, mg_ref[...], mb_ref[...]), 0.0)
    o_ref[...] = (
        jnp.dot(z, w2_ref[...], preferred_element_type=jnp.float32)
        + b2_ref[...]
        + h_ref[...]
    )


def _post(num, den, hn, h, W1, b1, mg, mb, W2, b2):
    return pl.pallas_call(
        _post_body,
        grid=(NB,),
        in_specs=[
            pl.BlockSpec((BR, F), lambda i: (i, 0)),
            pl.BlockSpec((BR, F), lambda i: (i, 0)),
            pl.BlockSpec((BR, F), lambda i: (i, 0)),
            pl.BlockSpec((BR, F), lambda i: (i, 0)),
            pl.BlockSpec((F, 2 * F), lambda i: (0, 0)),
            pl.BlockSpec((1, 2 * F), lambda i: (0, 0)),
            pl.BlockSpec((1, 2 * F), lambda i: (0, 0)),
            pl.BlockSpec((1, 2 * F), lambda i: (0, 0)),
            pl.BlockSpec((2 * F, F), lambda i: (0, 0)),
            pl.BlockSpec((1, F), lambda i: (0, 0)),
        ],
        out_specs=pl.BlockSpec((BR, F), lambda i: (i, 0)),
        out_shape=jax.ShapeDtypeStruct((N_NODES, F), jnp.float32),
    )(num, den, hn, h, W1, b1.reshape(1, 2 * F), mg.reshape(1, 2 * F),
      mb.reshape(1, 2 * F), W2, b2.reshape(1, F))


def _pool_body(h_ref, bat_ref, o_ref):
    @pl.when(pl.program_id(0) == 0)
    def _():
        o_ref[...] = jnp.zeros_like(o_ref)

    onehot = (bat_ref[...] ==
              lax.broadcasted_iota(jnp.int32, (1, N_GRAPHS), 1)).astype(jnp.float32)
    o_ref[...] += lax.dot_general(
        onehot, h_ref[...], (((0,), (0,)), ((), ())),
        preferred_element_type=jnp.float32)


def _pool(h, batch):
    return pl.pallas_call(
        _pool_body,
        grid=(NB,),
        in_specs=[
            pl.BlockSpec((BR, F), lambda i: (i, 0)),
            pl.BlockSpec((BR, 1), lambda i: (i, 0)),
        ],
        out_specs=pl.BlockSpec((N_GRAPHS, F), lambda i: (0, 0)),
        out_shape=jax.ShapeDtypeStruct((N_GRAPHS, F), jnp.float32),
    )(h, batch.reshape(N_NODES, 1))


def _part_body(src_hbm, dst_hbm, srcP_out, dstP_out, cnt_out,
               in_s, in_d, ob_s0, ob_d0, ob_s1, ob_d1, cnt_v, sem):
    c = lax.axis_index("c")
    s = lax.axis_index("s")

    @pl.when(c == 0)
    def _():
        zeros = jnp.zeros((16,), jnp.int32)
        sink = jnp.full((16,), SPLIT, jnp.int32)

        @pl.loop(0, CAP // 16)
        def _(r):
            ob_s0[pl.ds(r * 16, 16)] = zeros
            ob_s1[pl.ds(r * 16, 16)] = zeros
            ob_d0[pl.ds(r * 16, 16)] = sink
            ob_d1[pl.ds(r * 16, 16)] = sink

        def chunk(ch, carry):
            pltpu.sync_copy(src_hbm.at[s, ch], in_s)
            pltpu.sync_copy(dst_hbm.at[s, ch], in_d)
            o0, o1 = carry
            for q in range(CW // 16):
                dvec = in_d[pl.ds(q * 16, 16)]
                svec = in_s[pl.ds(q * 16, 16)]
                m0 = dvec < SPLIT
                n0 = plsc.all_reduce_population_count(m0)[0]
                plsc.store_compressed(ob_d0.at[pl.ds(o0, 16)], dvec, mask=m0)
                plsc.store_compressed(ob_s0.at[pl.ds(o0, 16)], svec, mask=m0)
                m1 = jnp.logical_not(m0)
                plsc.store_compressed(ob_d1.at[pl.ds(o1, 16)],
                                      dvec - SPLIT, mask=m1)
                plsc.store_compressed(ob_s1.at[pl.ds(o1, 16)], svec, mask=m1)
                o0 = o0 + n0
                o1 = o1 + (16 - n0)
            return o0, o1

        o0, o1 = pl.loop(0, NCHUNK,
                         init_carry=(jnp.int32(0), jnp.int32(0)))(chunk)

        lane = lax.iota(jnp.int32, 16)
        nch0 = (o0 + CW - 1) // CW
        nch1 = (o1 + CW - 1) // CW
        cnt_v[...] = jnp.where(lane == 0, nch0,
                               jnp.where(lane == 1, nch1, 0))
        pltpu.sync_copy(cnt_v, cnt_out.at[s])

        pltpu.sync_copy(ob_s0, srcP_out.at[s, 0])
        pltpu.sync_copy(ob_s1, srcP_out.at[s, 1])
        pltpu.sync_copy(ob_d0, dstP_out.at[s, 0])
        pltpu.sync_copy(ob_d1, dstP_out.at[s, 1])


@functools.cache
def _build_partition():
    return pl.kernel(
        _part_body,
        out_type=[
            jax.ShapeDtypeStruct((NS, 2, CAP), jnp.int32),
            jax.ShapeDtypeStruct((NS, 2, CAP), jnp.int32),
            jax.ShapeDtypeStruct((NS, 16), jnp.int32),
        ],
        mesh=plsc.VectorSubcoreMesh(core_axis_name="c", subcore_axis_name="s",
                                    num_cores=NC, num_subcores=NS),
        scratch_types=[
            pltpu.VMEM((CW,), jnp.int32),
            pltpu.VMEM((CW,), jnp.int32),
            pltpu.VMEM((CAP,), jnp.int32),
            pltpu.VMEM((CAP,), jnp.int32),
            pltpu.VMEM((CAP,), jnp.int32),
            pltpu.VMEM((CAP,), jnp.int32),
            pltpu.VMEM((16,), jnp.int32),
            pltpu.SemaphoreType.DMA,
        ],
        compiler_params=pltpu.CompilerParams(use_tc_tiling_on_sc=False,
                                             needs_layout_passes=False),
    )


def _agg_body(p_hbm, e_hbm, srcP_hbm, dstP_hbm, cnt_hbm, num_out, den_out,
              src_v, dst_v, rows_v, zbuf, cnt_v, acc, gsem0, gsem1):
    gsem = [gsem0, gsem1]
    c = lax.axis_index("c")
    s = lax.axis_index("s")

    pltpu.sync_copy(cnt_hbm.at[s], cnt_v)

    for r in range(16):
        for q in range(F // 16):
            zbuf[r, pl.ds(q * 16, 16)] = jnp.zeros((16,), jnp.float32)

    def half(h, tab, out_ref):
        pltpu.sync_copy(srcP_hbm.at[s, h], src_v)
        pltpu.sync_copy(dstP_hbm.at[s, h], dst_v)

        @pl.loop(0, SLAB // 16)
        def _(j):
            pltpu.sync_copy(zbuf, acc.at[pl.ds(s * SLAB + j * 16, 16)])

        plsc.subcore_barrier()

        nch = cnt_v[...][h]

        for b in range(2):
            @pl.when(b < nch)
            def _(b=b):
                pltpu.async_copy(tab.at[src_v.at[b]], rows_v.at[b], gsem[b])

        @pl.loop(0, (nch + 1) // 2)
        def _(q):
            for b in range(2):
                j = q * 2 + b

                @pl.when(j < nch)
                def _(j=j, b=b):
                    pltpu.make_async_copy(tab.at[src_v.at[j]], rows_v.at[b],
                                          gsem[b]).wait()
                    pltpu.sync_copy(rows_v.at[b], acc.at[dst_v.at[j]],
                                    add=True)

                    @pl.when(j + 2 < nch)
                    def _(j=j, b=b):
                        pltpu.async_copy(tab.at[src_v.at[j + 2]],
                                         rows_v.at[b], gsem[b])

        plsc.subcore_barrier()

        @pl.when(s < NS - 1)
        def _():
            pltpu.sync_copy(
                acc.at[pl.ds(s * SLAB, SLAB)],
                out_ref.at[pl.ds(h * SPLIT + s * SLAB, SLAB)])

        @pl.when(s == NS - 1)
        def _():
            pltpu.sync_copy(
                acc.at[pl.ds((NS - 1) * SLAB, LAST_SLAB)],
                out_ref.at[pl.ds(h * SPLIT + (NS - 1) * SLAB, LAST_SLAB)])

    def run(tab, out_ref):
        half(0, tab, out_ref)
        half(1, tab, out_ref)

    @pl.when(c == 0)
    def _():
        run(p_hbm, num_out)

    @pl.when(c == 1)
    def _():
        run(e_hbm, den_out)


@functools.cache
def _build_aggregate():
    return pl.kernel(
        _agg_body,
        out_type=[jax.ShapeDtypeStruct((N_NODES, F), jnp.float32)] * 2,
        mesh=plsc.VectorSubcoreMesh(core_axis_name="c", subcore_axis_name="s",
                                    num_cores=NC, num_subcores=NS),
        scratch_types=[
            pltpu.VMEM((NCHUNK, CW), jnp.int32),
            pltpu.VMEM((NCHUNK, CW), jnp.int32),
            pltpu.VMEM((2, CW, F), jnp.float32),
            pltpu.VMEM((16, F), jnp.float32),
            pltpu.VMEM((16,), jnp.int32),
            pltpu.VMEM_SHARED((ACC_ROWS, F), jnp.float32),
            pltpu.SemaphoreType.DMA,
            pltpu.SemaphoreType.DMA,
        ],
        compiler_params=pltpu.CompilerParams(use_tc_tiling_on_sc=False),
    )


def _sc_partition(srcI, dstI):
    return _build_partition()(srcI, dstI)


def _sc_aggregate(p, e, srcP, dstP, cnts):
    return _build_aggregate()(p, e, srcP, dstP, cnts)


def kernel(x, edge_index, batch, W0, b0, ln_g, ln_b, t, W1, b1, mg, mb,
           W2, b2):
    src = edge_index[0].astype(jnp.int32)
    dst = edge_index[1].astype(jnp.int32)
    srcI = jnp.pad(src, (0, E_PAD - N_EDGES)).reshape(NS, NCHUNK, CW)
    dstI = jnp.pad(dst, (0, E_PAD - N_EDGES),
                   constant_values=2 * SPLIT).reshape(NS, NCHUNK, CW)

    srcP, dstP, cnts = _sc_partition(srcI, dstI)
    srcP = srcP.reshape(NS, 2, NCHUNK, CW)
    dstP = dstP.reshape(NS, 2, NCHUNK, CW)

    h = _encode(x, W0, b0)
    for i in range(N_LAYERS):
        hn, p, e = _pre(h, ln_g[i], ln_b[i], t[i])
        num, den = _sc_aggregate(p, e, srcP, dstP, cnts)
        h = _post(num, den, hn, h, W1[i], b1[i], mg[i], mb[i], W2[i], b2[i])
    return _pool(h, batch.astype(jnp.int32))


# speedup vs baseline: 1.8131x; 1.0052x over previous
import functools

import jax
import jax.numpy as jnp
from jax import lax
from jax.experimental import pallas as pl
from jax.experimental.pallas import tpu as pltpu
from jax.experimental.pallas import tpu_sc as plsc

N_NODES = 10000
N_EDGES = 320000
F = 128
N_LAYERS = 6
N_GRAPHS = 16
EPS = 1e-7

NC = 2
NS = 16
CW = 96
EDGES_PT = -(-N_EDGES // (NS * CW)) * CW
NCHUNK = EDGES_PT // CW
E_PAD = EDGES_PT * NS
CAP = NCHUNK * CW

SPLIT = N_NODES // 2
ACC_ROWS = 5120
SLAB = ACC_ROWS // NS
LAST_SLAB = SPLIT - (NS - 1) * SLAB

BR = 2000
NB = N_NODES // BR


def _ln(z, g, b, eps=1e-5):
    m = jnp.mean(z, axis=-1, keepdims=True)
    v = jnp.mean((z - m) ** 2, axis=-1, keepdims=True)
    return (z - m) * lax.rsqrt(v + eps) * g + b


def _pre_math(h, g, b, t):
    hn = jnp.maximum(_ln(h, g, b), 0.0)
    m = hn + EPS
    e = jnp.exp(m * t)
    return hn, m * e, e


def _encpre_body(x_ref, w_ref, b_ref, g_ref, bb_ref, t_ref,
                 h_ref, hn_ref, p_ref, e_ref):
    h = (jnp.dot(x_ref[...], w_ref[...], preferred_element_type=jnp.float32)
         + b_ref[...])
    h_ref[...] = h
    hn, p, e = _pre_math(h, g_ref[...], bb_ref[...], t_ref[0, 0])
    hn_ref[...] = hn
    p_ref[...] = p
    e_ref[...] = e


def _encpre(x, W0, b0, g, b, t):
    return pl.pallas_call(
        _encpre_body,
        grid=(NB,),
        in_specs=[
            pl.BlockSpec((BR, F), lambda i: (i, 0)),
            pl.BlockSpec((F, F), lambda i: (0, 0)),
            pl.BlockSpec((1, F), lambda i: (0, 0)),
            pl.BlockSpec((1, F), lambda i: (0, 0)),
            pl.BlockSpec((1, F), lambda i: (0, 0)),
            pl.BlockSpec((1, 1), lambda i: (0, 0)),
        ],
        out_specs=[pl.BlockSpec((BR, F), lambda i: (i, 0))] * 4,
        out_shape=[jax.ShapeDtypeStruct((N_NODES, F), jnp.float32)] * 4,
    )(x, W0, b0.reshape(1, F), g.reshape(1, F), b.reshape(1, F),
      t.reshape(1, 1))


def _post_math(num_ref, den_ref, hn_ref, h_ref, w1_ref, b1_ref, mg_ref,
               mb_ref, w2_ref, b2_ref):
    den = den_ref[...]
    aggr = jnp.where(den > 0.0, num_ref[...] / den, 0.0)
    out = aggr + hn_ref[...]
    z = jnp.dot(out, w1_ref[...], preferred_element_type=jnp.float32) + b1_ref[...]
    z = jnp.maximum(_ln(z, mg_ref[...], mb_ref[...]), 0.0)
    return (jnp.dot(z, w2_ref[...], preferred_element_type=jnp.float32)
            + b2_ref[...] + h_ref[...])


_POST_SPECS = [
    pl.BlockSpec((BR, F), lambda i: (i, 0)),
    pl.BlockSpec((BR, F), lambda i: (i, 0)),
    pl.BlockSpec((BR, F), lambda i: (i, 0)),
    pl.BlockSpec((BR, F), lambda i: (i, 0)),
    pl.BlockSpec((F, 2 * F), lambda i: (0, 0)),
    pl.BlockSpec((1, 2 * F), lambda i: (0, 0)),
    pl.BlockSpec((1, 2 * F), lambda i: (0, 0)),
    pl.BlockSpec((1, 2 * F), lambda i: (0, 0)),
    pl.BlockSpec((2 * F, F), lambda i: (0, 0)),
    pl.BlockSpec((1, F), lambda i: (0, 0)),
]


def _postpre_body(num_ref, den_ref, hn_ref, h_ref, w1_ref, b1_ref, mg_ref,
                  mb_ref, w2_ref, b2_ref, g_ref, bb_ref, t_ref,
                  h2_ref, hn2_ref, p_ref, e_ref):
    h2 = _post_math(num_ref, den_ref, hn_ref, h_ref, w1_ref, b1_ref,
                    mg_ref, mb_ref, w2_ref, b2_ref)
    h2_ref[...] = h2
    hn2, p, e = _pre_math(h2, g_ref[...], bb_ref[...], t_ref[0, 0])
    hn2_ref[...] = hn2
    p_ref[...] = p
    e_ref[...] = e


def _postpre(num, den, hn, h, W1, b1, mg, mb, W2, b2, g, b, t):
    return pl.pallas_call(
        _postpre_body,
        grid=(NB,),
        in_specs=_POST_SPECS + [
            pl.BlockSpec((1, F), lambda i: (0, 0)),
            pl.BlockSpec((1, F), lambda i: (0, 0)),
            pl.BlockSpec((1, 1), lambda i: (0, 0)),
        ],
        out_specs=[pl.BlockSpec((BR, F), lambda i: (i, 0))] * 4,
        out_shape=[jax.ShapeDtypeStruct((N_NODES, F), jnp.float32)] * 4,
    )(num, den, hn, h, W1, b1.reshape(1, 2 * F), mg.reshape(1, 2 * F),
      mb.reshape(1, 2 * F), W2, b2.reshape(1, F), g.reshape(1, F),
      b.reshape(1, F), t.reshape(1, 1))


def _postpool_body(num_ref, den_ref, hn_ref, h_ref, w1_ref, b1_ref, mg_ref,
                   mb_ref, w2_ref, b2_ref, bat_ref, o_ref):
    h2 = _post_math(num_ref, den_ref, hn_ref, h_ref, w1_ref, b1_ref,
                    mg_ref, mb_ref, w2_ref, b2_ref)

    @pl.when(pl.program_id(0) == 0)
    def _():
        o_ref[...] = jnp.zeros_like(o_ref)

    onehot = (bat_ref[...] ==
              lax.broadcasted_iota(jnp.int32, (1, N_GRAPHS), 1)).astype(jnp.float32)
    o_ref[...] += lax.dot_general(
        onehot, h2, (((0,), (0,)), ((), ())),
        preferred_element_type=jnp.float32)


def _postpool(num, den, hn, h, W1, b1, mg, mb, W2, b2, batch):
    return pl.pallas_call(
        _postpool_body,
        grid=(NB,),
        in_specs=_POST_SPECS + [pl.BlockSpec((BR, 1), lambda i: (i, 0))],
        out_specs=pl.BlockSpec((N_GRAPHS, F), lambda i: (0, 0)),
        out_shape=jax.ShapeDtypeStruct((N_GRAPHS, F), jnp.float32),
    )(num, den, hn, h, W1, b1.reshape(1, 2 * F), mg.reshape(1, 2 * F),
      mb.reshape(1, 2 * F), W2, b2.reshape(1, F), batch.reshape(N_NODES, 1))


def _part_body(src_hbm, dst_hbm, srcP_out, dstP_out, cnt_out,
               in_s, in_d, ob_s0, ob_d0, ob_s1, ob_d1, cnt_v, sem):
    c = lax.axis_index("c")
    s = lax.axis_index("s")

    @pl.when(c == 0)
    def _():
        zeros = jnp.zeros((16,), jnp.int32)
        sink = jnp.full((16,), SPLIT, jnp.int32)

        @pl.loop(0, CAP // 16)
        def _(r):
            ob_s0[pl.ds(r * 16, 16)] = zeros
            ob_s1[pl.ds(r * 16, 16)] = zeros
            ob_d0[pl.ds(r * 16, 16)] = sink
            ob_d1[pl.ds(r * 16, 16)] = sink

        def chunk(ch, carry):
            pltpu.sync_copy(src_hbm.at[s, ch], in_s)
            pltpu.sync_copy(dst_hbm.at[s, ch], in_d)
            o0, o1 = carry
            for q in range(CW // 16):
                dvec = in_d[pl.ds(q * 16, 16)]
                svec = in_s[pl.ds(q * 16, 16)]
                m0 = dvec < SPLIT
                n0 = plsc.all_reduce_population_count(m0)[0]
                plsc.store_compressed(ob_d0.at[pl.ds(o0, 16)], dvec, mask=m0)
                plsc.store_compressed(ob_s0.at[pl.ds(o0, 16)], svec, mask=m0)
                m1 = jnp.logical_not(m0)
                plsc.store_compressed(ob_d1.at[pl.ds(o1, 16)],
                                      dvec - SPLIT, mask=m1)
                plsc.store_compressed(ob_s1.at[pl.ds(o1, 16)], svec, mask=m1)
                o0 = o0 + n0
                o1 = o1 + (16 - n0)
            return o0, o1

        o0, o1 = pl.loop(0, NCHUNK,
                         init_carry=(jnp.int32(0), jnp.int32(0)))(chunk)

        lane = lax.iota(jnp.int32, 16)
        nch0 = (o0 + CW - 1) // CW
        nch1 = (o1 + CW - 1) // CW
        cnt_v[...] = jnp.where(lane == 0, nch0,
                               jnp.where(lane == 1, nch1, 0))
        pltpu.sync_copy(cnt_v, cnt_out.at[s])

        pltpu.sync_copy(ob_s0, srcP_out.at[s, 0])
        pltpu.sync_copy(ob_s1, srcP_out.at[s, 1])
        pltpu.sync_copy(ob_d0, dstP_out.at[s, 0])
        pltpu.sync_copy(ob_d1, dstP_out.at[s, 1])


@functools.cache
def _build_partition():
    return pl.kernel(
        _part_body,
        out_type=[
            jax.ShapeDtypeStruct((NS, 2, CAP), jnp.int32),
            jax.ShapeDtypeStruct((NS, 2, CAP), jnp.int32),
            jax.ShapeDtypeStruct((NS, 16), jnp.int32),
        ],
        mesh=plsc.VectorSubcoreMesh(core_axis_name="c", subcore_axis_name="s",
                                    num_cores=NC, num_subcores=NS),
        scratch_types=[
            pltpu.VMEM((CW,), jnp.int32),
            pltpu.VMEM((CW,), jnp.int32),
            pltpu.VMEM((CAP,), jnp.int32),
            pltpu.VMEM((CAP,), jnp.int32),
            pltpu.VMEM((CAP,), jnp.int32),
            pltpu.VMEM((CAP,), jnp.int32),
            pltpu.VMEM((16,), jnp.int32),
            pltpu.SemaphoreType.DMA,
        ],
        compiler_params=pltpu.CompilerParams(use_tc_tiling_on_sc=False,
                                             needs_layout_passes=False),
    )


def _agg_body(p_hbm, e_hbm, srcP_hbm, dstP_hbm, cnt_hbm, num_out, den_out,
              src_v, dst_v, rows_v, zbuf, cnt_v, acc, gsem0, gsem1):
    gsem = [gsem0, gsem1]
    c = lax.axis_index("c")
    s = lax.axis_index("s")

    pltpu.sync_copy(cnt_hbm.at[s], cnt_v)

    for r in range(16):
        for q in range(F // 16):
            zbuf[r, pl.ds(q * 16, 16)] = jnp.zeros((16,), jnp.float32)

    def half(h, tab, out_ref):
        pltpu.sync_copy(srcP_hbm.at[s, h], src_v)
        pltpu.sync_copy(dstP_hbm.at[s, h], dst_v)

        @pl.loop(0, SLAB // 16)
        def _(j):
            pltpu.sync_copy(zbuf, acc.at[pl.ds(s * SLAB + j * 16, 16)])

        plsc.subcore_barrier()

        nch = cnt_v[...][h]

        for b in range(2):
            @pl.when(b < nch)
            def _(b=b):
                pltpu.async_copy(tab.at[src_v.at[b]], rows_v.at[b], gsem[b])

        @pl.loop(0, (nch + 1) // 2)
        def _(q):
            for b in range(2):
                j = q * 2 + b

                @pl.when(j < nch)
                def _(j=j, b=b):
                    pltpu.make_async_copy(tab.at[src_v.at[j]], rows_v.at[b],
                                          gsem[b]).wait()
                    pltpu.sync_copy(rows_v.at[b], acc.at[dst_v.at[j]],
                                    add=True)

                    @pl.when(j + 2 < nch)
                    def _(j=j, b=b):
                        pltpu.async_copy(tab.at[src_v.at[j + 2]],
                                         rows_v.at[b], gsem[b])

        plsc.subcore_barrier()

        @pl.when(s < NS - 1)
        def _():
            pltpu.sync_copy(
                acc.at[pl.ds(s * SLAB, SLAB)],
                out_ref.at[pl.ds(h * SPLIT + s * SLAB, SLAB)])

        @pl.when(s == NS - 1)
        def _():
            pltpu.sync_copy(
                acc.at[pl.ds((NS - 1) * SLAB, LAST_SLAB)],
                out_ref.at[pl.ds(h * SPLIT + (NS - 1) * SLAB, LAST_SLAB)])

    def run(tab, out_ref):
        half(0, tab, out_ref)
        half(1, tab, out_ref)

    @pl.when(c == 0)
    def _():
        run(p_hbm, num_out)

    @pl.when(c == 1)
    def _():
        run(e_hbm, den_out)


@functools.cache
def _build_aggregate():
    return pl.kernel(
        _agg_body,
        out_type=[jax.ShapeDtypeStruct((N_NODES, F), jnp.float32)] * 2,
        mesh=plsc.VectorSubcoreMesh(core_axis_name="c", subcore_axis_name="s",
                                    num_cores=NC, num_subcores=NS),
        scratch_types=[
            pltpu.VMEM((NCHUNK, CW), jnp.int32),
            pltpu.VMEM((NCHUNK, CW), jnp.int32),
            pltpu.VMEM((2, CW, F), jnp.float32),
            pltpu.VMEM((16, F), jnp.float32),
            pltpu.VMEM((16,), jnp.int32),
            pltpu.VMEM_SHARED((ACC_ROWS, F), jnp.float32),
            pltpu.SemaphoreType.DMA,
            pltpu.SemaphoreType.DMA,
        ],
        compiler_params=pltpu.CompilerParams(use_tc_tiling_on_sc=False),
    )


def _sc_partition(srcI, dstI):
    return _build_partition()(srcI, dstI)


def _sc_aggregate(p, e, srcP, dstP, cnts):
    return _build_aggregate()(p, e, srcP, dstP, cnts)


def kernel(x, edge_index, batch, W0, b0, ln_g, ln_b, t, W1, b1, mg, mb,
           W2, b2):
    src = edge_index[0].astype(jnp.int32)
    dst = edge_index[1].astype(jnp.int32)
    srcI = jnp.pad(src, (0, E_PAD - N_EDGES)).reshape(NS, NCHUNK, CW)
    dstI = jnp.pad(dst, (0, E_PAD - N_EDGES),
                   constant_values=2 * SPLIT).reshape(NS, NCHUNK, CW)

    srcP, dstP, cnts = _sc_partition(srcI, dstI)
    srcP = srcP.reshape(NS, 2, NCHUNK, CW)
    dstP = dstP.reshape(NS, 2, NCHUNK, CW)

    h, hn, p, e = _encpre(x, W0, b0, ln_g[0], ln_b[0], t[0])
    for i in range(N_LAYERS - 1):
        num, den = _sc_aggregate(p, e, srcP, dstP, cnts)
        h, hn, p, e = _postpre(num, den, hn, h, W1[i], b1[i], mg[i], mb[i],
                               W2[i], b2[i], ln_g[i + 1], ln_b[i + 1],
                               t[i + 1])
    num, den = _sc_aggregate(p, e, srcP, dstP, cnts)
    i = N_LAYERS - 1
    return _postpool(num, den, hn, h, W1[i], b1[i], mg[i], mb[i], W2[i],
                     b2[i], batch.astype(jnp.int32))
